# baseline (speedup 1.0000x reference)
import jax
import jax.numpy as jnp
from jax import lax
from jax.experimental import pallas as pl
from jax.experimental.pallas import tpu as pltpu
from jax.experimental.pallas import tpu_sc as plsc

N_CAND = 1024
SEQ = 128
D = 256
NUM_FEAT = 32

_INFO = plsc.get_sparse_core_info()
_NC = _INFO.num_cores
_NS = _INFO.num_subcores
_NW = _NC * _NS
_HALF = N_CAND // 2
_BPW = _HALF // _NW
_L = 16

_BL_COLS = (1, 2, 4, 5)
_AT_COLS = (0, 3)
_OUT_W = (len(_BL_COLS) + len(_AT_COLS)) * D + NUM_FEAT


def _make_body(half):
  def _body(bl_hbm, at_hbm, ct_hbm, nf_hbm, out_hbm,
            c_v, bl_idx, at_idx, bl_buf, at_buf, nf_v, csem, gsem, wsem):
    wid = lax.axis_index("s") * _NC + lax.axis_index("c")
    base = half * _HALF + wid * _BPW

    blk = base // 128
    off = base % 128
    ccopies = []
    for f, col in enumerate(_BL_COLS + _AT_COLS):
      ccopies.append(pltpu.async_copy(
          ct_hbm.at[col, blk, pl.ds(off, _BPW)], c_v[f], csem))
    for cc in ccopies:
      cc.wait()

    lanes = lax.iota(jnp.int32, _L)
    for f, _col in enumerate(_BL_COLS):
      for h in range(_BPW // _L):
        rows16 = lanes + (h * _L)
        cvals = c_v[f][pl.ds(h * _L, _L)]
        bl_idx[pl.ds(f * _BPW + h * _L, _L)] = (base + rows16) * SEQ + cvals
    for f, _col in enumerate(_AT_COLS):
      for h in range(_BPW // _L):
        rows16 = lanes + (h * _L)
        cvals = c_v[len(_BL_COLS) + f][pl.ds(h * _L, _L)]
        at_idx[pl.ds(f * _BPW + h * _L, _L)] = (
            (cvals + 2) * N_CAND + base + rows16)

    blc = pltpu.async_copy(bl_hbm.at[bl_idx], bl_buf, gsem)
    atc = pltpu.async_copy(at_hbm.at[at_idx], at_buf, gsem)

    obase = wid * _BPW
    pltpu.sync_copy(nf_hbm.at[pl.ds(base, _BPW)], nf_v)
    pltpu.sync_copy(
        nf_v,
        out_hbm.at[pl.ds(obase, _BPW), pl.ds(len(_BL_COLS + _AT_COLS) * D,
                                             NUM_FEAT)])

    wcopies = []
    blc.wait()
    for f in range(len(_BL_COLS)):
      wcopies.append(pltpu.async_copy(
          bl_buf.at[pl.ds(f * _BPW, _BPW)],
          out_hbm.at[pl.ds(obase, _BPW), pl.ds(f * D, D)], wsem))
    atc.wait()
    for f in range(len(_AT_COLS)):
      wcopies.append(pltpu.async_copy(
          at_buf.at[pl.ds(f * _BPW, _BPW)],
          out_hbm.at[pl.ds(obase, _BPW), pl.ds((len(_BL_COLS) + f) * D, D)],
          wsem))
    for wc in wcopies:
      wc.wait()

  return _body


@jax.jit
def _run(bl_flat, at_flat, cflat, nf):
  mesh = plsc.VectorSubcoreMesh(core_axis_name="c", subcore_axis_name="s")
  nbl = len(_BL_COLS) * _BPW
  nat = len(_AT_COLS) * _BPW
  scratch = [
      [pltpu.VMEM((_BPW,), jnp.int32) for _ in range(6)],
      pltpu.VMEM((nbl,), jnp.int32),
      pltpu.VMEM((nat,), jnp.int32),
      pltpu.VMEM((nbl, D), jnp.float32),
      pltpu.VMEM((nat, D), jnp.float32),
      pltpu.VMEM((_BPW, NUM_FEAT), jnp.float32),
      pltpu.SemaphoreType.DMA,
      pltpu.SemaphoreType.DMA,
      pltpu.SemaphoreType.DMA,
  ]
  halves = []
  for half in (0, 1):
    fn = pl.kernel(
        _make_body(half),
        out_type=jax.ShapeDtypeStruct((_HALF, _OUT_W), jnp.float32),
        mesh=mesh,
        scratch_types=scratch,
    )
    halves.append(fn(bl_flat, at_flat, cflat, nf))
  return jnp.concatenate(halves, axis=0)


def kernel(candidates, candidate_numeric_features, stacked_bi_lstm_output,
           stacked_attended_nodes):
  cflat = candidates[0].T.reshape(6, N_CAND // 128, 128)
  nf = candidate_numeric_features[0]
  bl_flat = stacked_bi_lstm_output.reshape(N_CAND * SEQ, D)
  at_flat = stacked_attended_nodes[0].transpose(1, 0, 2).reshape(
      (SEQ + 2) * N_CAND, D)
  return _run(bl_flat, at_flat, cflat, nf)[None]


# speedup vs baseline: 1.2748x; 1.2748x over previous
import jax
import jax.numpy as jnp
from jax import lax
from jax.experimental import pallas as pl
from jax.experimental.pallas import tpu as pltpu
from jax.experimental.pallas import tpu_sc as plsc

N_CAND = 1024
SEQ = 128
D = 256
NUM_FEAT = 32

_INFO = plsc.get_sparse_core_info()
_NC = _INFO.num_cores
_NS = _INFO.num_subcores
_NW = _NC * _NS
_BPW = N_CAND // _NW
_L = 16

_BL_COLS = (1, 2, 4, 5)
_AT_COLS = (0, 3)
_GATHER_W = (len(_BL_COLS) + len(_AT_COLS)) * D


def _body(bl_hbm, at_hbm, ct_hbm, nf_hbm, out_hbm,
          c_v, bl_idx, at_idx, bl_buf, at_buf, nf_v, csem, gsem, wsem):
  wid = lax.axis_index("s") * _NC + lax.axis_index("c")
  base = wid * _BPW

  ccopies = []
  for f, col in enumerate(_BL_COLS + _AT_COLS):
    ccopies.append(pltpu.async_copy(
        ct_hbm.at[pl.ds(col * N_CAND + base, _BPW)], c_v[f], csem))
  for cc in ccopies:
    cc.wait()

  lanes = lax.iota(jnp.int32, _L)
  for f, _col in enumerate(_BL_COLS):
    for h in range(_BPW // _L):
      rows16 = lanes + (h * _L)
      cvals = c_v[f][pl.ds(h * _L, _L)]
      bl_idx[pl.ds(f * _BPW + h * _L, _L)] = (base + rows16) * SEQ + cvals
  for f, _col in enumerate(_AT_COLS):
    for h in range(_BPW // _L):
      rows16 = lanes + (h * _L)
      cvals = c_v[len(_BL_COLS) + f][pl.ds(h * _L, _L)]
      at_idx[pl.ds(f * _BPW + h * _L, _L)] = (
          (cvals + 2) * N_CAND + base + rows16)

  blc = pltpu.async_copy(bl_hbm.at[bl_idx], bl_buf, gsem)
  atc = pltpu.async_copy(at_hbm.at[at_idx], at_buf, gsem)

  pltpu.sync_copy(nf_hbm.at[pl.ds(base, _BPW)], nf_v)
  pltpu.sync_copy(
      nf_v, out_hbm.at[pl.ds(base, _BPW), pl.ds(len(_BL_COLS + _AT_COLS) * D,
                                                NUM_FEAT)])

  wcopies = []
  blc.wait()
  for f in range(len(_BL_COLS)):
    wcopies.append(pltpu.async_copy(
        bl_buf.at[pl.ds(f * _BPW, _BPW)],
        out_hbm.at[pl.ds(base, _BPW), pl.ds(f * D, D)], wsem))
  atc.wait()
  for f in range(len(_AT_COLS)):
    wcopies.append(pltpu.async_copy(
        at_buf.at[pl.ds(f * _BPW, _BPW)],
        out_hbm.at[pl.ds(base, _BPW), pl.ds((len(_BL_COLS) + f) * D, D)],
        wsem))
  for wc in wcopies:
    wc.wait()


@jax.jit
def _run(bl_flat, at_flat, cflat, nf):
  mesh = plsc.VectorSubcoreMesh(core_axis_name="c", subcore_axis_name="s")
  nbl = len(_BL_COLS) * _BPW
  nat = len(_AT_COLS) * _BPW
  scratch = [
      [pltpu.VMEM((_BPW,), jnp.int32) for _ in range(6)],
      pltpu.VMEM((nbl,), jnp.int32),
      pltpu.VMEM((nat,), jnp.int32),
      pltpu.VMEM((nbl, D), jnp.float32),
      pltpu.VMEM((nat, D), jnp.float32),
      pltpu.VMEM((_BPW, NUM_FEAT), jnp.float32),
      pltpu.SemaphoreType.DMA,
      pltpu.SemaphoreType.DMA,
      pltpu.SemaphoreType.DMA,
  ]
  fn = pl.kernel(
      _body,
      out_type=jax.ShapeDtypeStruct((N_CAND, _GATHER_W + NUM_FEAT),
                                    jnp.float32),
      mesh=mesh,
      scratch_types=scratch,
  )
  return fn(bl_flat, at_flat, cflat, nf)


def kernel(candidates, candidate_numeric_features, stacked_bi_lstm_output,
           stacked_attended_nodes):
  cflat = candidates[0].T.reshape(6 * N_CAND)
  nf = candidate_numeric_features[0]
  bl_flat = stacked_bi_lstm_output.reshape(N_CAND * SEQ, D)
  at_flat = stacked_attended_nodes[0].transpose(1, 0, 2).reshape(
      (SEQ + 2) * N_CAND, D)
  return _run(bl_flat, at_flat, cflat, nf)[None]
